# baseline (speedup 1.0000x reference)
import functools

import jax
import jax.numpy as jnp
from jax import lax
from jax.experimental import pallas as pl
from jax.experimental.pallas import tpu as pltpu

BR = 128


def _gat_body(Z_ref, A_ref, W_ref, as_ref, ar_ref, out_ref, C_ref,
              H_ref, Hb_ref, f1_ref, f2_ref):
    i = pl.program_id(0)

    @pl.when(i == 0)
    def _prologue():
        H = jnp.dot(Z_ref[...], W_ref[...], preferred_element_type=jnp.float32)
        H_ref[...] = H
        Hb_ref[...] = H.astype(jnp.bfloat16)
        f1_ref[...] = 0.5 * lax.dot_general(
            H, as_ref[...], (((1,), (1,)), ((), ())),
            preferred_element_type=jnp.float32)
        f2_ref[...] = 0.5 * lax.dot_general(
            ar_ref[...], H, (((1,), (1,)), ((), ())),
            preferred_element_type=jnp.float32)

    f1b = f1_ref[pl.ds(i * BR, BR), :]
    e = 0.5 * jnp.tanh(f1b + f2_ref[...]) + 0.5
    E = A_ref[...] * e
    r = 1.0 / (jnp.sum(E, axis=1, keepdims=True) + 1e-8)
    C = E * r
    C_ref[...] = C
    out_ref[...] = jnp.dot(C.astype(jnp.bfloat16), Hb_ref[...],
                           preferred_element_type=jnp.float32)


def _gat(Z, A, W, a_s, a_r):
    N = A.shape[0]
    din, dout = W.shape
    out, C = pl.pallas_call(
        _gat_body,
        grid=(N // BR,),
        in_specs=[
            pl.BlockSpec((N, din), lambda i: (0, 0)),
            pl.BlockSpec((BR, N), lambda i: (i, 0)),
            pl.BlockSpec((din, dout), lambda i: (0, 0)),
            pl.BlockSpec((1, dout), lambda i: (0, 0)),
            pl.BlockSpec((1, dout), lambda i: (0, 0)),
        ],
        out_specs=(
            pl.BlockSpec((BR, dout), lambda i: (i, 0)),
            pl.BlockSpec((BR, N), lambda i: (i, 0)),
        ),
        out_shape=(
            jax.ShapeDtypeStruct((N, dout), jnp.float32),
            jax.ShapeDtypeStruct((N, N), jnp.float32),
        ),
        scratch_shapes=[
            pltpu.VMEM((N, dout), jnp.float32),
            pltpu.VMEM((N, dout), jnp.bfloat16),
            pltpu.VMEM((N, 1), jnp.float32),
            pltpu.VMEM((1, N), jnp.float32),
        ],
    )(Z, A, W, a_s.reshape(1, -1), a_r.reshape(1, -1))
    return out, C


def _gat_an_body(Z_ref, A_ref, W_ref, as_ref, ar_ref, P_ref,
                 out_ref, C_ref, An_ref, H_ref, Hb_ref, f1_ref, f2_ref):
    i = pl.program_id(0)

    @pl.when(i == 0)
    def _prologue():
        H = jnp.dot(Z_ref[...], W_ref[...], preferred_element_type=jnp.float32)
        H_ref[...] = H
        Hb_ref[...] = H.astype(jnp.bfloat16)
        f1_ref[...] = 0.5 * lax.dot_general(
            H, as_ref[...], (((1,), (1,)), ((), ())),
            preferred_element_type=jnp.float32)
        f2_ref[...] = 0.5 * lax.dot_general(
            ar_ref[...], H, (((1,), (1,)), ((), ())),
            preferred_element_type=jnp.float32)

    A_blk = A_ref[...]
    f1b = f1_ref[pl.ds(i * BR, BR), :]
    e = 0.5 * jnp.tanh(f1b + f2_ref[...]) + 0.5
    E = A_blk * e
    r = 1.0 / (jnp.sum(E, axis=1, keepdims=True) + 1e-8)
    C = E * r
    C_ref[...] = C
    out_ref[...] = jnp.dot(C.astype(jnp.bfloat16), Hb_ref[...],
                           preferred_element_type=jnp.float32)

    AP = jnp.dot(A_blk, P_ref[...], preferred_element_type=jnp.float32)
    Pb = P_ref[pl.ds(i * BR, BR), :]
    contrib = lax.dot_general(Pb, AP, (((0,), (0,)), ((), ())),
                              preferred_element_type=jnp.float32)

    @pl.when(i == 0)
    def _init():
        An_ref[...] = contrib

    @pl.when(i > 0)
    def _acc():
        An_ref[...] += contrib


def _gat_with_an(Z, A, W, a_s, a_r, P):
    N = A.shape[0]
    din, dout = W.shape
    c = P.shape[1]
    out, C, An = pl.pallas_call(
        _gat_an_body,
        grid=(N // BR,),
        in_specs=[
            pl.BlockSpec((N, din), lambda i: (0, 0)),
            pl.BlockSpec((BR, N), lambda i: (i, 0)),
            pl.BlockSpec((din, dout), lambda i: (0, 0)),
            pl.BlockSpec((1, dout), lambda i: (0, 0)),
            pl.BlockSpec((1, dout), lambda i: (0, 0)),
            pl.BlockSpec((N, c), lambda i: (0, 0)),
        ],
        out_specs=(
            pl.BlockSpec((BR, dout), lambda i: (i, 0)),
            pl.BlockSpec((BR, N), lambda i: (i, 0)),
            pl.BlockSpec((c, c), lambda i: (0, 0)),
        ),
        out_shape=(
            jax.ShapeDtypeStruct((N, dout), jnp.float32),
            jax.ShapeDtypeStruct((N, N), jnp.float32),
            jax.ShapeDtypeStruct((c, c), jnp.float32),
        ),
        scratch_shapes=[
            pltpu.VMEM((N, dout), jnp.float32),
            pltpu.VMEM((N, dout), jnp.bfloat16),
            pltpu.VMEM((N, 1), jnp.float32),
            pltpu.VMEM((1, N), jnp.float32),
        ],
    )(Z, A, W, a_s.reshape(1, -1), a_r.reshape(1, -1), P)
    return out, C, An


def _ahat_body(Z_ref, g_ref, b_ref, out_ref):
    i = pl.program_id(0)
    Zb = Z_ref[pl.ds(i * BR, BR), :]
    G = lax.dot_general(Zb, Z_ref[...], (((1,), (1,)), ((), ())),
                        preferred_element_type=jnp.float32)
    mu = jnp.mean(G, axis=1, keepdims=True)
    d = G - mu
    var = jnp.mean(d * d, axis=1, keepdims=True)
    y = d * lax.rsqrt(var + 1e-5) * g_ref[...] + b_ref[...]
    out_ref[...] = 0.5 * jnp.tanh(0.5 * y) + 0.5


def _ahat(Z, g, b):
    N = Z.shape[0]
    h = Z.shape[1]
    return pl.pallas_call(
        _ahat_body,
        grid=(N // BR,),
        in_specs=[
            pl.BlockSpec((N, h), lambda i: (0, 0)),
            pl.BlockSpec((1, N), lambda i: (0, 0)),
            pl.BlockSpec((1, N), lambda i: (0, 0)),
        ],
        out_specs=pl.BlockSpec((BR, N), lambda i: (i, 0)),
        out_shape=jax.ShapeDtypeStruct((N, N), jnp.float32),
    )(Z, g.reshape(1, -1), b.reshape(1, -1))


def kernel(X, A, params):
    Z = X
    enc_attn = []
    for li in range(3):
        Z, C = _gat(Z, A, params['We%d' % li], params['ase%d' % li],
                    params['are%d' % li])
        enc_attn.append(C)

    A_hat = _ahat(Z, params['g_ln'], params['b_ln'])

    P0 = jax.nn.softmax(Z @ params['Wc0'] + params['bc0'], axis=1)
    S0 = jnp.argmax(P0, axis=1)

    dec_attn = []
    Xd, C, An1 = _gat_with_an(Z, A, params['Wd0'], params['asd0'],
                              params['ard0'], P0)
    dec_attn.append(C)
    for li in range(1, 3):
        Xd, C = _gat(Xd, A, params['Wd%d' % li], params['asd%d' % li],
                     params['ard%d' % li])
        dec_attn.append(C)
    X_hat = Xd

    Xn1 = P0.T @ Z

    P1 = jax.nn.softmax(Xn1 @ params['Wc1'] + params['bc1'], axis=1)
    S1 = jnp.argmax(P1, axis=1)
    Xn2 = P1.T @ Xn1
    An2 = P1.T @ An1 @ P1

    X_all_final = [Z, Xn1, Xn2]
    A_all_final = [A, An1, An2]
    P_all = [P0, P1]
    S_all = [S0, S1]
    return (X_hat, A_hat, X_all_final, A_all_final, P_all, S_all,
            [enc_attn, dec_attn])


# speedup vs baseline: 1.2009x; 1.2009x over previous
import functools

import jax
import jax.numpy as jnp
from jax import lax
from jax.experimental import pallas as pl
from jax.experimental.pallas import tpu as pltpu

BR = 512


def _gat_body(Z_ref, A_ref, W_ref, as_ref, ar_ref, out_ref, C_ref,
              H_ref, Hb_ref, f1_ref, f2_ref):
    i = pl.program_id(0)

    @pl.when(i == 0)
    def _prologue():
        H = jnp.dot(Z_ref[...], W_ref[...], preferred_element_type=jnp.float32)
        H_ref[...] = H
        Hb_ref[...] = H.astype(jnp.bfloat16)
        f1_ref[...] = 0.5 * lax.dot_general(
            H, as_ref[...], (((1,), (1,)), ((), ())),
            preferred_element_type=jnp.float32)
        f2_ref[...] = 0.5 * lax.dot_general(
            ar_ref[...], H, (((1,), (1,)), ((), ())),
            preferred_element_type=jnp.float32)

    f1b = f1_ref[pl.ds(i * BR, BR), :]
    e = 0.5 * jnp.tanh(f1b + f2_ref[...]) + 0.5
    E = A_ref[...] * e
    r = 1.0 / (jnp.sum(E, axis=1, keepdims=True) + 1e-8)
    C = E * r
    C_ref[...] = C
    out_ref[...] = jnp.dot(C.astype(jnp.bfloat16), Hb_ref[...],
                           preferred_element_type=jnp.float32)


def _gat(Z, A, W, a_s, a_r):
    N = A.shape[0]
    din, dout = W.shape
    out, C = pl.pallas_call(
        _gat_body,
        grid=(N // BR,),
        in_specs=[
            pl.BlockSpec((N, din), lambda i: (0, 0)),
            pl.BlockSpec((BR, N), lambda i: (i, 0)),
            pl.BlockSpec((din, dout), lambda i: (0, 0)),
            pl.BlockSpec((1, dout), lambda i: (0, 0)),
            pl.BlockSpec((1, dout), lambda i: (0, 0)),
        ],
        out_specs=(
            pl.BlockSpec((BR, dout), lambda i: (i, 0)),
            pl.BlockSpec((BR, N), lambda i: (i, 0)),
        ),
        out_shape=(
            jax.ShapeDtypeStruct((N, dout), jnp.float32),
            jax.ShapeDtypeStruct((N, N), jnp.float32),
        ),
        scratch_shapes=[
            pltpu.VMEM((N, dout), jnp.float32),
            pltpu.VMEM((N, dout), jnp.bfloat16),
            pltpu.VMEM((N, 1), jnp.float32),
            pltpu.VMEM((1, N), jnp.float32),
        ],
    )(Z, A, W, a_s.reshape(1, -1), a_r.reshape(1, -1))
    return out, C


def _gat_an_body(Z_ref, A_ref, W_ref, as_ref, ar_ref, P_ref,
                 out_ref, C_ref, An_ref, H_ref, Hb_ref, f1_ref, f2_ref):
    i = pl.program_id(0)

    @pl.when(i == 0)
    def _prologue():
        H = jnp.dot(Z_ref[...], W_ref[...], preferred_element_type=jnp.float32)
        H_ref[...] = H
        Hb_ref[...] = H.astype(jnp.bfloat16)
        f1_ref[...] = 0.5 * lax.dot_general(
            H, as_ref[...], (((1,), (1,)), ((), ())),
            preferred_element_type=jnp.float32)
        f2_ref[...] = 0.5 * lax.dot_general(
            ar_ref[...], H, (((1,), (1,)), ((), ())),
            preferred_element_type=jnp.float32)

    A_blk = A_ref[...]
    f1b = f1_ref[pl.ds(i * BR, BR), :]
    e = 0.5 * jnp.tanh(f1b + f2_ref[...]) + 0.5
    E = A_blk * e
    r = 1.0 / (jnp.sum(E, axis=1, keepdims=True) + 1e-8)
    C = E * r
    C_ref[...] = C
    out_ref[...] = jnp.dot(C.astype(jnp.bfloat16), Hb_ref[...],
                           preferred_element_type=jnp.float32)

    AP = jnp.dot(A_blk, P_ref[...], preferred_element_type=jnp.float32)
    Pb = P_ref[pl.ds(i * BR, BR), :]
    contrib = lax.dot_general(Pb, AP, (((0,), (0,)), ((), ())),
                              preferred_element_type=jnp.float32)

    @pl.when(i == 0)
    def _init():
        An_ref[...] = contrib

    @pl.when(i > 0)
    def _acc():
        An_ref[...] += contrib


def _gat_with_an(Z, A, W, a_s, a_r, P):
    N = A.shape[0]
    din, dout = W.shape
    c = P.shape[1]
    out, C, An = pl.pallas_call(
        _gat_an_body,
        grid=(N // BR,),
        in_specs=[
            pl.BlockSpec((N, din), lambda i: (0, 0)),
            pl.BlockSpec((BR, N), lambda i: (i, 0)),
            pl.BlockSpec((din, dout), lambda i: (0, 0)),
            pl.BlockSpec((1, dout), lambda i: (0, 0)),
            pl.BlockSpec((1, dout), lambda i: (0, 0)),
            pl.BlockSpec((N, c), lambda i: (0, 0)),
        ],
        out_specs=(
            pl.BlockSpec((BR, dout), lambda i: (i, 0)),
            pl.BlockSpec((BR, N), lambda i: (i, 0)),
            pl.BlockSpec((c, c), lambda i: (0, 0)),
        ),
        out_shape=(
            jax.ShapeDtypeStruct((N, dout), jnp.float32),
            jax.ShapeDtypeStruct((N, N), jnp.float32),
            jax.ShapeDtypeStruct((c, c), jnp.float32),
        ),
        scratch_shapes=[
            pltpu.VMEM((N, dout), jnp.float32),
            pltpu.VMEM((N, dout), jnp.bfloat16),
            pltpu.VMEM((N, 1), jnp.float32),
            pltpu.VMEM((1, N), jnp.float32),
        ],
    )(Z, A, W, a_s.reshape(1, -1), a_r.reshape(1, -1), P)
    return out, C, An


def _ahat_body(Z_ref, g_ref, b_ref, out_ref):
    i = pl.program_id(0)
    Zb = Z_ref[pl.ds(i * BR, BR), :]
    G = lax.dot_general(Zb, Z_ref[...], (((1,), (1,)), ((), ())),
                        preferred_element_type=jnp.float32)
    mu = jnp.mean(G, axis=1, keepdims=True)
    d = G - mu
    var = jnp.mean(d * d, axis=1, keepdims=True)
    y = d * lax.rsqrt(var + 1e-5) * g_ref[...] + b_ref[...]
    out_ref[...] = 0.5 * jnp.tanh(0.5 * y) + 0.5


def _ahat(Z, g, b):
    N = Z.shape[0]
    h = Z.shape[1]
    return pl.pallas_call(
        _ahat_body,
        grid=(N // BR,),
        in_specs=[
            pl.BlockSpec((N, h), lambda i: (0, 0)),
            pl.BlockSpec((1, N), lambda i: (0, 0)),
            pl.BlockSpec((1, N), lambda i: (0, 0)),
        ],
        out_specs=pl.BlockSpec((BR, N), lambda i: (i, 0)),
        out_shape=jax.ShapeDtypeStruct((N, N), jnp.float32),
    )(Z, g.reshape(1, -1), b.reshape(1, -1))


def kernel(X, A, params):
    Z = X
    enc_attn = []
    for li in range(3):
        Z, C = _gat(Z, A, params['We%d' % li], params['ase%d' % li],
                    params['are%d' % li])
        enc_attn.append(C)

    A_hat = _ahat(Z, params['g_ln'], params['b_ln'])

    P0 = jax.nn.softmax(Z @ params['Wc0'] + params['bc0'], axis=1)
    S0 = jnp.argmax(P0, axis=1)

    dec_attn = []
    Xd, C, An1 = _gat_with_an(Z, A, params['Wd0'], params['asd0'],
                              params['ard0'], P0)
    dec_attn.append(C)
    for li in range(1, 3):
        Xd, C = _gat(Xd, A, params['Wd%d' % li], params['asd%d' % li],
                     params['ard%d' % li])
        dec_attn.append(C)
    X_hat = Xd

    Xn1 = P0.T @ Z

    P1 = jax.nn.softmax(Xn1 @ params['Wc1'] + params['bc1'], axis=1)
    S1 = jnp.argmax(P1, axis=1)
    Xn2 = P1.T @ Xn1
    An2 = P1.T @ An1 @ P1

    X_all_final = [Z, Xn1, Xn2]
    A_all_final = [A, An1, An2]
    P_all = [P0, P1]
    S_all = [S0, S1]
    return (X_hat, A_hat, X_all_final, A_all_final, P_all, S_all,
            [enc_attn, dec_attn])
